# Initial kernel scaffold; baseline (speedup 1.0000x reference)
import jax
import jax.numpy as jnp
from jax.experimental import pallas as pl


def kernel(z, edge_index):
    raise NotImplementedError("write your pallas kernel here")


# speedup vs baseline: 2.6915x; 2.6915x over previous
import functools

import jax
import jax.numpy as jnp
from jax import lax
from jax.experimental import pallas as pl
from jax.experimental.pallas import tpu as pltpu
from jax.experimental.pallas import tpu_sc as plsc

_GATHER_DNUMS = lax.GatherDimensionNumbers(
    offset_dims=(), collapsed_slice_dims=(0,), start_index_map=(0,))


def _shuffle(t, idx):
    return lax.gather(t, idx[:, None], _GATHER_DNUMS, slice_sizes=(1,),
                      mode=lax.GatherScatterMode.PROMISE_IN_BOUNDS)


D = 128
L = 16
CHUNK = 80
NC, NS = 2, 16
NW = NC * NS


def _make_sc_call(E):
    e_per = E // NW
    n_chunks = e_per // CHUNK
    mesh = plsc.VectorSubcoreMesh(core_axis_name="c", subcore_axis_name="s")

    @functools.partial(
        pl.kernel,
        out_type=jax.ShapeDtypeStruct((E,), jnp.float32),
        mesh=mesh,
        scratch_types=[
            pltpu.VMEM((CHUNK,), jnp.int32),
            pltpu.VMEM((CHUNK,), jnp.int32),
            pltpu.VMEM((CHUNK, D), jnp.float32),
            pltpu.VMEM((CHUNK, D), jnp.float32),
            pltpu.VMEM((CHUNK,), jnp.float32),
            pltpu.SemaphoreType.DMA,
        ],
    )
    def sc_call(z_hbm, src_hbm, dst_hbm, out_hbm, idx_s, idx_d, rows_s,
                rows_d, out_v, sem):
        wid = lax.axis_index("s") * NC + lax.axis_index("c")
        base = wid * e_per
        lane = lax.broadcasted_iota(jnp.int32, (L,), 0)

        def chunk_body(ci, _):
            off = base + ci * CHUNK
            pltpu.sync_copy(src_hbm.at[pl.ds(off, CHUNK)], idx_s)
            pltpu.sync_copy(dst_hbm.at[pl.ds(off, CHUNK)], idx_d)
            cp_s = pltpu.async_copy(z_hbm.at[idx_s], rows_s, sem)
            cp_d = pltpu.async_copy(z_hbm.at[idx_d], rows_d, sem)
            cp_s.wait()
            cp_d.wait()

            def group_body(g, _):
                acc = jnp.zeros((L,), jnp.float32)
                for j in range(L):
                    e = g * L + j
                    t = rows_s[e, pl.ds(0, L)] * rows_d[e, pl.ds(0, L)]
                    for k in range(1, D // L):
                        t = t + (rows_s[e, pl.ds(k * L, L)]
                                 * rows_d[e, pl.ds(k * L, L)])
                    for sh in (8, 4, 2, 1):
                        t = t + _shuffle(t, lane ^ sh)
                    acc = jnp.where(lane == j, t, acc)
                out_v[pl.ds(g * L, L)] = 1.0 / (1.0 + jnp.exp(-acc))
                return 0

            lax.fori_loop(0, CHUNK // L, group_body, 0)
            pltpu.sync_copy(out_v, out_hbm.at[pl.ds(off, CHUNK)])
            return 0

        lax.fori_loop(0, n_chunks, chunk_body, 0)

    return sc_call


def kernel(z, edge_index):
    E = edge_index.shape[1]
    ei = edge_index.astype(jnp.int32)
    return _make_sc_call(E)(z, ei[0], ei[1])


# speedup vs baseline: 4.8219x; 1.7916x over previous
import functools

import jax
import jax.numpy as jnp
from jax import lax
from jax.experimental import pallas as pl
from jax.experimental.pallas import tpu as pltpu
from jax.experimental.pallas import tpu_sc as plsc

_GATHER_DNUMS = lax.GatherDimensionNumbers(
    offset_dims=(), collapsed_slice_dims=(0,), start_index_map=(0,))


def _shuffle(t, idx):
    return lax.gather(t, idx[:, None], _GATHER_DNUMS, slice_sizes=(1,),
                      mode=lax.GatherScatterMode.PROMISE_IN_BOUNDS)


D = 128
L = 16
CHUNK = 80
NC, NS = 2, 16
NW = NC * NS


def _make_sc_call(E):
    n_chunks = E // CHUNK
    cpw = n_chunks // NW
    npairs = cpw // 2
    mesh = plsc.VectorSubcoreMesh(core_axis_name="c", subcore_axis_name="s")

    @functools.partial(
        pl.kernel,
        out_type=jax.ShapeDtypeStruct((NW, cpw, CHUNK), jnp.float32),
        mesh=mesh,
        scratch_types=[
            pltpu.VMEM((cpw, CHUNK), jnp.int32),
            pltpu.VMEM((cpw, CHUNK), jnp.int32),
            pltpu.VMEM((CHUNK, D), jnp.float32),
            pltpu.VMEM((CHUNK, D), jnp.float32),
            pltpu.VMEM((CHUNK, D), jnp.float32),
            pltpu.VMEM((CHUNK, D), jnp.float32),
            pltpu.VMEM((cpw, CHUNK), jnp.float32),
            pltpu.SemaphoreType.DMA,
            pltpu.SemaphoreType.DMA,
        ],
    )
    def sc_call(z_hbm, src_hbm, dst_hbm, out_hbm, idx_s, idx_d,
                rs_a, rd_a, rs_b, rd_b, out_v, sem_a, sem_b):
        wid = lax.axis_index("s") * NC + lax.axis_index("c")
        lane = lax.broadcasted_iota(jnp.int32, (L,), 0)

        pltpu.sync_copy(src_hbm.at[wid], idx_s)
        pltpu.sync_copy(dst_hbm.at[wid], idx_d)

        def fire(ci, rs, rd, sem):
            pltpu.async_copy(z_hbm.at[idx_s.at[ci]], rs, sem)
            pltpu.async_copy(z_hbm.at[idx_d.at[ci]], rd, sem)

        def drain(ci, rs, rd, sem):
            pltpu.make_async_copy(z_hbm.at[idx_s.at[ci]], rs, sem).wait()
            pltpu.make_async_copy(z_hbm.at[idx_d.at[ci]], rd, sem).wait()

        def compute(ci, rs, rd):
            def group_body(g, _):
                acc = jnp.zeros((L,), jnp.float32)
                for j in range(L):
                    e = g * L + j
                    t = rs[e, pl.ds(0, L)] * rd[e, pl.ds(0, L)]
                    for k in range(1, D // L):
                        t = t + rs[e, pl.ds(k * L, L)] * rd[e, pl.ds(k * L, L)]
                    for sh in (8, 4, 2, 1):
                        t = t + _shuffle(t, lane ^ sh)
                    acc = jnp.where(lane == j, t, acc)
                out_v[ci, pl.ds(g * L, L)] = 1.0 / (1.0 + jnp.exp(-acc))
                return 0

            lax.fori_loop(0, CHUNK // L, group_body, 0)

        fire(0, rs_a, rd_a, sem_a)

        def pair_body(pi, _):
            ca = 2 * pi
            cb = ca + 1
            fire(cb, rs_b, rd_b, sem_b)
            drain(ca, rs_a, rd_a, sem_a)
            compute(ca, rs_a, rd_a)
            fire(ca + 2, rs_a, rd_a, sem_a)
            drain(cb, rs_b, rd_b, sem_b)
            compute(cb, rs_b, rd_b)
            return 0

        lax.fori_loop(0, npairs, pair_body, 0)
        drain(cpw - 1, rs_a, rd_a, sem_a)
        compute(cpw - 1, rs_a, rd_a)

        pltpu.sync_copy(out_v, out_hbm.at[wid])

    return sc_call


def kernel(z, edge_index):
    E = edge_index.shape[1]
    ei = edge_index.astype(jnp.int32)
    cpw = E // CHUNK // NW
    src3d = ei[0].reshape(NW, cpw, CHUNK)
    dst3d = ei[1].reshape(NW, cpw, CHUNK)
    out3d = _make_sc_call(E)(z, src3d, dst3d)
    return out3d.reshape(E)
